# baseline (speedup 1.0000x reference)
import jax
import jax.numpy as jnp
import numpy as np
from jax.experimental import pallas as pl
from jax.experimental.pallas import tpu as pltpu

N_T, N_Y, N_X = 8, 32, 32
NB = N_X * N_Y
OFFS = (-64, -33, -32, -31, -2, -1, 0, 1, 2, 31, 32, 33, 64)
S = (-32, -1, 0, 1, 32)
N_BLK = 3 * N_T - 2
ROW_TILE = 512


def _np_masks():
    k = np.arange(NB)
    x = k % N_X
    y = k // N_X
    me = ((x + 1) < N_X).astype(np.float32)
    mw = ((x - 1) >= 0).astype(np.float32)
    mn = ((y + 1) < N_Y).astype(np.float32)
    ms = ((y - 1) >= 0).astype(np.float32)
    return me, mw, mn, ms


_ME, _MW, _MN, _MS = _np_masks()


def _shift_lanes(v, o):
    if o == 0:
        return v
    z = jnp.zeros((1, abs(o)), v.dtype)
    if o > 0:
        return jnp.concatenate([z, v[:, : NB - o]], axis=1)
    return jnp.concatenate([v[:, -o:], z], axis=1)


SUB = 128
ROW_SLAB = 512


def _band_kernel(a_ref, b_ref, w_ref, e_ref, out_ref):
    rt = pl.program_id(2)
    w = w_ref[0, 0]
    dd = {d: None for d in OFFS}
    for i1, o1 in enumerate(S):
        aw = a_ref[0, 0, i1 : i1 + 1, :] * w
        for i2, o2 in enumerate(S):
            term = _shift_lanes(aw * b_ref[0, 0, i2 : i2 + 1, :], o1)
            d = o2 - o1
            dd[d] = term if dd[d] is None else dd[d] + term
    dd[0] = dd[0] + e_ref[0, 0]
    g = {d: _shift_lanes(dd[d], d) for d in OFFS}
    nsub = NB // SUB
    nsub_loc = ROW_SLAB // SUB
    jr = jax.lax.broadcasted_iota(jnp.int32, (SUB, SUB), 1) - jax.lax.broadcasted_iota(
        jnp.int32, (SUB, SUB), 0
    )
    zero = jnp.zeros((SUB, SUB), jnp.float32)

    def emit(slab):
        for sr_loc in range(nsub_loc):
            sr = slab * nsub_loc + sr_loc
            r0 = sr_loc * SUB
            for sc in range(nsub):
                delta = sc - sr
                if abs(delta) > 1:
                    out_ref[0, 0, r0 : r0 + SUB, sc * SUB : (sc + 1) * SUB] = zero
                    continue
                acc = zero
                for d in OFFS:
                    c = d - SUB * delta
                    if c <= -SUB or c >= SUB:
                        continue
                    gd = g[d][:, sc * SUB : (sc + 1) * SUB]
                    acc = jnp.where(jr == c, jnp.broadcast_to(gd, (SUB, SUB)), acc)
                out_ref[0, 0, r0 : r0 + SUB, sc * SUB : (sc + 1) * SUB] = acc

    for slab in range(NB // ROW_SLAB):
        @pl.when(rt == slab)
        def _(slab=slab):
            emit(slab)


def kernel(kappa, m, H, tau):
    del H
    kap = kappa[0]
    t = jnp.squeeze(tau, axis=1)
    qt = jnp.transpose(1.0 / (t * t), (0, 2, 1))
    m1 = jnp.transpose(m[:, 0], (0, 2, 1))
    m2 = jnp.transpose(m[:, 1], (0, 2, 1))
    u1 = 0.5 * m1 * _ME
    l1 = -0.5 * m1 * _MW
    u32 = 0.5 * m2 * _MN
    l32 = -0.5 * m2 * _MS
    k2 = kap * kap
    dvec = jnp.concatenate(
        [jnp.full((2, 1, NB), k2), jnp.full((2, N_T - 1, NB), 1.0 + k2)], axis=1
    )
    Md = jnp.stack([l32, l1, dvec, u1, u32], axis=2)

    ones = jnp.ones((2, NB), jnp.float32)
    zcol = jnp.zeros((2, NB), jnp.float32)
    e0 = jnp.zeros((2, 5, NB), jnp.float32).at[:, 2, :].set(1.0)

    A_l, B_l, W_l, E_l = [], [], [], []

    def add(a, b, w, e):
        A_l.append(a)
        B_l.append(b)
        W_l.append(w)
        E_l.append(e)

    add(Md[:, 0], Md[:, 0], ones, 1.05 * ones)
    add(e0, Md[:, 1], -qt[:, 1], zcol)
    for i in range(1, N_T - 1):
        add(Md[:, i], e0, -qt[:, i], zcol)
        add(Md[:, i], Md[:, i], qt[:, i], qt[:, i])
        add(e0, Md[:, i + 1], -qt[:, i + 1], zcol)
    add(Md[:, N_T - 1], e0, -qt[:, N_T - 1], zcol)
    add(Md[:, N_T - 1], Md[:, N_T - 1], qt[:, N_T - 1], zcol)

    A = jnp.stack(A_l, axis=1)
    B = jnp.stack(B_l, axis=1)
    W = jnp.stack(W_l, axis=1)[:, :, None, :]
    E = jnp.stack(E_l, axis=1)[:, :, None, :]

    return pl.pallas_call(
        _band_kernel,
        grid=(2, N_BLK, NB // ROW_SLAB),
        in_specs=[
            pl.BlockSpec((1, 1, 5, NB), lambda b, k, r: (b, k, 0, 0)),
            pl.BlockSpec((1, 1, 5, NB), lambda b, k, r: (b, k, 0, 0)),
            pl.BlockSpec((1, 1, 1, NB), lambda b, k, r: (b, k, 0, 0)),
            pl.BlockSpec((1, 1, 1, NB), lambda b, k, r: (b, k, 0, 0)),
        ],
        out_specs=pl.BlockSpec((1, 1, ROW_SLAB, NB), lambda b, k, r: (b, k, r, 0)),
        out_shape=jax.ShapeDtypeStruct((2, N_BLK, NB, NB), jnp.float32),
        compiler_params=pltpu.CompilerParams(
            dimension_semantics=("parallel", "parallel", "arbitrary")
        ),
    )(A, B, W, E)


# speedup vs baseline: 1.4491x; 1.4491x over previous
import jax
import jax.numpy as jnp
import numpy as np
from jax.experimental import pallas as pl
from jax.experimental.pallas import tpu as pltpu

N_T, N_Y, N_X = 8, 32, 32
NB = N_X * N_Y
OFFS = (-64, -33, -32, -31, -2, -1, 0, 1, 2, 31, 32, 33, 64)
S = (-32, -1, 0, 1, 32)
N_BLK = 3 * N_T - 2
ROW_TILE = 512


def _np_masks():
    k = np.arange(NB)
    x = k % N_X
    y = k // N_X
    me = ((x + 1) < N_X).astype(np.float32)
    mw = ((x - 1) >= 0).astype(np.float32)
    mn = ((y + 1) < N_Y).astype(np.float32)
    ms = ((y - 1) >= 0).astype(np.float32)
    return me, mw, mn, ms


_ME, _MW, _MN, _MS = _np_masks()


def _shift_lanes(v, o):
    if o == 0:
        return v
    z = jnp.zeros((1, abs(o)), v.dtype)
    if o > 0:
        return jnp.concatenate([z, v[:, : NB - o]], axis=1)
    return jnp.concatenate([v[:, -o:], z], axis=1)


PER_STEP = 2
SUB = 128


def _band_kernel(a_ref, b_ref, w_ref, e_ref, out_ref):
  for kk in range(PER_STEP):
      w = w_ref[0, kk]
      dd = {d: None for d in OFFS}
      for i1, o1 in enumerate(S):
          aw = a_ref[0, kk, i1 : i1 + 1, :] * w
          for i2, o2 in enumerate(S):
              term = _shift_lanes(aw * b_ref[0, kk, i2 : i2 + 1, :], o1)
              d = o2 - o1
              dd[d] = term if dd[d] is None else dd[d] + term
      dd[0] = dd[0] + e_ref[0, kk]
      g = {d: _shift_lanes(dd[d], d) for d in OFFS}
      nsub = NB // SUB
      jr = jax.lax.broadcasted_iota(jnp.int32, (SUB, SUB), 1) - jax.lax.broadcasted_iota(
          jnp.int32, (SUB, SUB), 0
      )
      zero = jnp.zeros((SUB, SUB), jnp.float32)
      for sr in range(nsub):
          for sc in range(nsub):
              delta = sc - sr
              if abs(delta) > 1:
                  out_ref[0, kk, sr * SUB : (sr + 1) * SUB, sc * SUB : (sc + 1) * SUB] = zero
                  continue
              acc = zero
              for d in OFFS:
                  c = d - SUB * delta
                  if c <= -SUB or c >= SUB:
                      continue
                  gd = g[d][:, sc * SUB : (sc + 1) * SUB]
                  acc = jnp.where(jr == c, jnp.broadcast_to(gd, (SUB, SUB)), acc)
              out_ref[0, kk, sr * SUB : (sr + 1) * SUB, sc * SUB : (sc + 1) * SUB] = acc


def kernel(kappa, m, H, tau):
    del H
    kap = kappa[0]
    t = jnp.squeeze(tau, axis=1)
    qt = jnp.transpose(1.0 / (t * t), (0, 2, 1))
    m1 = jnp.transpose(m[:, 0], (0, 2, 1))
    m2 = jnp.transpose(m[:, 1], (0, 2, 1))
    u1 = 0.5 * m1 * _ME
    l1 = -0.5 * m1 * _MW
    u32 = 0.5 * m2 * _MN
    l32 = -0.5 * m2 * _MS
    k2 = kap * kap
    dvec = jnp.concatenate(
        [jnp.full((2, 1, NB), k2), jnp.full((2, N_T - 1, NB), 1.0 + k2)], axis=1
    )
    Md = jnp.stack([l32, l1, dvec, u1, u32], axis=2)

    ones = jnp.ones((2, NB), jnp.float32)
    zcol = jnp.zeros((2, NB), jnp.float32)
    e0 = jnp.zeros((2, 5, NB), jnp.float32).at[:, 2, :].set(1.0)

    A_l, B_l, W_l, E_l = [], [], [], []

    def add(a, b, w, e):
        A_l.append(a)
        B_l.append(b)
        W_l.append(w)
        E_l.append(e)

    add(Md[:, 0], Md[:, 0], ones, 1.05 * ones)
    add(e0, Md[:, 1], -qt[:, 1], zcol)
    for i in range(1, N_T - 1):
        add(Md[:, i], e0, -qt[:, i], zcol)
        add(Md[:, i], Md[:, i], qt[:, i], qt[:, i])
        add(e0, Md[:, i + 1], -qt[:, i + 1], zcol)
    add(Md[:, N_T - 1], e0, -qt[:, N_T - 1], zcol)
    add(Md[:, N_T - 1], Md[:, N_T - 1], qt[:, N_T - 1], zcol)

    A = jnp.stack(A_l, axis=1)
    B = jnp.stack(B_l, axis=1)
    W = jnp.stack(W_l, axis=1)[:, :, None, :]
    E = jnp.stack(E_l, axis=1)[:, :, None, :]

    return pl.pallas_call(
        _band_kernel,
        grid=(2, N_BLK // PER_STEP),
        in_specs=[
            pl.BlockSpec((1, PER_STEP, 5, NB), lambda b, k: (b, k, 0, 0)),
            pl.BlockSpec((1, PER_STEP, 5, NB), lambda b, k: (b, k, 0, 0)),
            pl.BlockSpec((1, PER_STEP, 1, NB), lambda b, k: (b, k, 0, 0)),
            pl.BlockSpec((1, PER_STEP, 1, NB), lambda b, k: (b, k, 0, 0)),
        ],
        out_specs=pl.BlockSpec((1, PER_STEP, NB, NB), lambda b, k: (b, k, 0, 0)),
        out_shape=jax.ShapeDtypeStruct((2, N_BLK, NB, NB), jnp.float32),
        compiler_params=pltpu.CompilerParams(
            dimension_semantics=("parallel", "parallel")
        ),
    )(A, B, W, E)
